# baseline (speedup 1.0000x reference)
import functools

import jax
import jax.numpy as jnp
import numpy as np
from jax import lax
from jax.experimental import pallas as pl
from jax.experimental.pallas import tpu as pltpu
from jax.experimental.pallas import tpu_sc as plsc

_E = 16
_G = 4
_T = 12 * 2048
_CAP = 2048.0
_ROWS = 96
_ROW_TOKENS = 1024
_ROW_WORDS = _ROW_TOKENS * _E
_BLOCKS = _ROW_TOKENS // _E

_info = plsc.get_sparse_core_info()
_NC, _NS = _info.num_cores, _info.num_subcores
_NW = _NC * _NS
_ROWS_PER_W = _ROWS // _NW
_ACC = _ROWS_PER_W * _E * _E

_LN2 = 0.6931471805599453
_SQRT2 = 1.4142135623730951


def _vlog(s):
    bits = lax.bitcast_convert_type(s, jnp.int32)
    e = jnp.right_shift(bits, 23) - 127
    mant = jnp.bitwise_or(jnp.bitwise_and(bits, 0x007FFFFF), 0x3F800000)
    f = lax.bitcast_convert_type(mant, jnp.float32)
    big = f >= _SQRT2
    f = jnp.where(big, f * 0.5, f)
    e = e + jnp.where(big, 1, 0)
    t = (f - 1.0) / (f + 1.0)
    t2 = t * t
    w = 2.0 * t * (1.0 + t2 * (1.0 / 3.0 + t2 * (0.2 + t2 * (1.0 / 7.0 + t2 / 9.0))))
    return e.astype(jnp.float32) * _LN2 + w


def _tree(f, xs):
    while len(xs) > 1:
        nxt = [f(xs[i], xs[i + 1]) for i in range(0, len(xs) - 1, 2)]
        if len(xs) % 2:
            nxt.append(xs[-1])
        xs = nxt
    return xs[0]


def _block(buf, base, iota, iota16, zvec, off, pacc, cacc):
    l = []
    for e in range(_E):
        rot = jnp.bitwise_and(iota + e, _E - 1)
        addr = iota16 + rot + base
        l.append(plsc.load_gather(buf, [addr]))
    m = _tree(jnp.maximum, l)
    ex = [jnp.exp(le - m) for le in l]
    s = _tree(lambda a, b: a + b, ex)
    r = 1.0 / s
    for e in range(_E):
        plsc.addupdate(pacc.at[pl.ds(off + e * _E, _E)], ex[e] * r)
        plsc.addupdate(cacc.at[pl.ds(off + e * _E, _E)],
                       jnp.where(l[e] == m, 1.0, 0.0))
    logz = m + _vlog(s)
    return zvec + logz * logz


def _row_body(buf, j, iota, iota16, pacc, cacc, zvec):
    off = j * _E * _E

    def body(k, z):
        return _block(buf, k * (_E * _E), iota, iota16, z, off, pacc, cacc)

    return lax.fori_loop(0, _BLOCKS, body, zvec)


@functools.partial(
    pl.kernel,
    out_type=(
        jax.ShapeDtypeStruct((_NW, _ACC), jnp.float32),
        jax.ShapeDtypeStruct((_NW, _ACC), jnp.float32),
        jax.ShapeDtypeStruct((_NW, _E), jnp.float32),
    ),
    mesh=plsc.VectorSubcoreMesh(core_axis_name="c", subcore_axis_name="s"),
    compiler_params=pltpu.CompilerParams(needs_layout_passes=False),
    scratch_types=[
        pltpu.VMEM((_ROW_WORDS,), jnp.float32),
        pltpu.VMEM((_ROW_WORDS,), jnp.float32),
        pltpu.VMEM((_ACC,), jnp.float32),
        pltpu.VMEM((_ACC,), jnp.float32),
        pltpu.VMEM((_E,), jnp.float32),
        pltpu.SemaphoreType.DMA,
        pltpu.SemaphoreType.DMA,
    ],
)
def _router_loss_sc(x_hbm, p_out, c_out, z_out,
                    buf0, buf1, pacc, cacc, zacc, sem0, sem1):
    wid = lax.axis_index("s") * _NC + lax.axis_index("c")
    iota = lax.iota(jnp.int32, _E)
    iota16 = iota * _E
    zero = jnp.zeros((_E,), jnp.float32)
    for i in range(_ACC // _E):
        pacc[pl.ds(i * _E, _E)] = zero
        cacc[pl.ds(i * _E, _E)] = zero

    r0 = wid * _ROWS_PER_W
    cp0 = pltpu.async_copy(x_hbm.at[r0], buf0, sem0)
    cp1 = pltpu.async_copy(x_hbm.at[r0 + 1], buf1, sem1)
    cp0.wait()
    z0 = _row_body(buf0, 0, iota, iota16, pacc, cacc, zero)
    cp2 = pltpu.async_copy(x_hbm.at[r0 + 2], buf0, sem0)
    cp1.wait()
    z1 = _row_body(buf1, 1, iota, iota16, pacc, cacc, z0)
    cp2.wait()
    z2 = _row_body(buf0, 2, iota, iota16, pacc, cacc, z1)

    zacc[...] = z2
    pltpu.sync_copy(pacc, p_out.at[wid])
    pltpu.sync_copy(cacc, c_out.at[wid])
    pltpu.sync_copy(zacc, z_out.at[wid])


_UNROT = np.equal((np.arange(_E)[:, None] + np.arange(_E)[None, :]) % _E,
                  np.arange(_E)[None, None, :]).astype(np.float32)
_rows = np.arange(_NW * _ROWS_PER_W).reshape(_NW, _ROWS_PER_W)
_GSEL = np.equal(((_rows // 2) % _G)[:, :, None],
                 np.arange(_G)[None, None, :]).astype(np.float32)


def kernel(router_outputs, attention_mask):
    del attention_mask
    x = router_outputs.reshape(_ROWS, _ROW_WORDS)
    p_parts, c_parts, z_parts = _router_loss_sc(x)
    p4 = p_parts.reshape(_NW, _ROWS_PER_W, _E, _E)
    c4 = c_parts.reshape(_NW, _ROWS_PER_W, _E, _E)
    probs = jnp.einsum("wjet,etE,wjg->gE", p4, _UNROT, _GSEL)
    counts = jnp.einsum("wjet,etE,wjg->gE", c4, _UNROT, _GSEL)
    z_loss = z_parts.sum() / (_G * _T)
    clipped = jnp.minimum(counts, _CAP)
    overflow = jnp.sum(counts - clipped, axis=1)
    final_c = clipped.at[:, 0].add(overflow)
    aux = jnp.mean((final_c / _T) * (probs / _T)) * _E ** 2
    return (0.001 * z_loss + 0.001 * aux).astype(jnp.float32)


# speedup vs baseline: 1.1641x; 1.1641x over previous
import functools

import jax
import jax.numpy as jnp
import numpy as np
from jax import lax
from jax.experimental import pallas as pl
from jax.experimental.pallas import tpu as pltpu
from jax.experimental.pallas import tpu_sc as plsc

_E = 16
_G = 4
_T = 12 * 2048
_CAP = 2048.0
_ROWS = 96
_ROW_TOKENS = 1024
_ROW_WORDS = _ROW_TOKENS * _E
_BLOCKS = _ROW_TOKENS // _E

_info = plsc.get_sparse_core_info()
_NC, _NS = _info.num_cores, _info.num_subcores
_NW = _NC * _NS
_ROWS_PER_W = _ROWS // _NW
_ACC = _ROWS_PER_W * _E * _E

_LN2 = 0.6931471805599453
_SQRT2 = 1.4142135623730951


def _vlog(s):
    bits = lax.bitcast_convert_type(s, jnp.int32)
    e = jnp.right_shift(bits, 23) - 127
    mant = jnp.bitwise_or(jnp.bitwise_and(bits, 0x007FFFFF), 0x3F800000)
    f = lax.bitcast_convert_type(mant, jnp.float32)
    big = f >= _SQRT2
    f = jnp.where(big, f * 0.5, f)
    e = e + jnp.where(big, 1, 0)
    t = (f - 1.0) / (f + 1.0)
    t2 = t * t
    w = 2.0 * t * (1.0 + t2 * (1.0 / 3.0 + t2 * (0.2 + t2 * (1.0 / 7.0 + t2 / 9.0))))
    return e.astype(jnp.float32) * _LN2 + w


def _tree(f, xs):
    while len(xs) > 1:
        nxt = [f(xs[i], xs[i + 1]) for i in range(0, len(xs) - 1, 2)]
        if len(xs) % 2:
            nxt.append(xs[-1])
        xs = nxt
    return xs[0]


def _block(buf, base, iota, iota16, zvec, off, pacc, cacc):
    del iota16
    tok = iota + base
    l = []
    for e in range(_E):
        rot = jnp.bitwise_and(iota + e, _E - 1)
        l.append(plsc.load_gather(buf, [tok, rot]))
    m = _tree(jnp.maximum, l)
    ex = [jnp.exp(le - m) for le in l]
    s = _tree(lambda a, b: a + b, ex)
    r = 1.0 / s
    for e in range(_E):
        plsc.addupdate(pacc.at[pl.ds(off + e * _E, _E)], ex[e] * r)
        plsc.addupdate(cacc.at[pl.ds(off + e * _E, _E)],
                       jnp.where(l[e] == m, 1.0, 0.0))
    logz = m + _vlog(s)
    return zvec + logz * logz


def _row_body(buf, j, iota, iota16, pacc, cacc, zvec):
    off = j * _E * _E

    def body(k, z):
        return _block(buf, k * _E, iota, iota16, z, off, pacc, cacc)

    return lax.fori_loop(0, _BLOCKS, body, zvec)


@functools.partial(
    pl.kernel,
    out_type=(
        jax.ShapeDtypeStruct((_NW, _ACC), jnp.float32),
        jax.ShapeDtypeStruct((_NW, _ACC), jnp.float32),
        jax.ShapeDtypeStruct((_NW, _E), jnp.float32),
    ),
    mesh=plsc.VectorSubcoreMesh(core_axis_name="c", subcore_axis_name="s"),
    compiler_params=pltpu.CompilerParams(needs_layout_passes=False,
                                         use_tc_tiling_on_sc=False),
    scratch_types=[
        pltpu.VMEM((_ROW_TOKENS, _E), jnp.float32),
        pltpu.VMEM((_ROW_TOKENS, _E), jnp.float32),
        pltpu.VMEM((_ACC,), jnp.float32),
        pltpu.VMEM((_ACC,), jnp.float32),
        pltpu.VMEM((_E,), jnp.float32),
        pltpu.SemaphoreType.DMA,
        pltpu.SemaphoreType.DMA,
    ],
)
def _router_loss_sc(x_hbm, p_out, c_out, z_out,
                    buf0, buf1, pacc, cacc, zacc, sem0, sem1):
    wid = lax.axis_index("s") * _NC + lax.axis_index("c")
    iota = lax.iota(jnp.int32, _E)
    iota16 = iota * _E
    zero = jnp.zeros((_E,), jnp.float32)
    for i in range(_ACC // _E):
        pacc[pl.ds(i * _E, _E)] = zero
        cacc[pl.ds(i * _E, _E)] = zero

    r0 = wid * _ROWS_PER_W

    def row_src(r):
        i, g, h = r // 8, (r // 2) % _G, r % 2
        return x_hbm.at[i, g, pl.ds(h * _ROW_TOKENS, _ROW_TOKENS), :]

    cp0 = pltpu.async_copy(row_src(r0), buf0, sem0)
    cp1 = pltpu.async_copy(row_src(r0 + 1), buf1, sem1)
    cp0.wait()
    z0 = _row_body(buf0, 0, iota, iota16, pacc, cacc, zero)
    cp2 = pltpu.async_copy(row_src(r0 + 2), buf0, sem0)
    cp1.wait()
    z1 = _row_body(buf1, 1, iota, iota16, pacc, cacc, z0)
    cp2.wait()
    z2 = _row_body(buf0, 2, iota, iota16, pacc, cacc, z1)

    zacc[...] = z2
    pltpu.sync_copy(pacc, p_out.at[wid])
    pltpu.sync_copy(cacc, c_out.at[wid])
    pltpu.sync_copy(zacc, z_out.at[wid])


_UNROT = np.equal((np.arange(_E)[:, None] + np.arange(_E)[None, :]) % _E,
                  np.arange(_E)[None, None, :]).astype(np.float32)
_rows = np.arange(_NW * _ROWS_PER_W).reshape(_NW, _ROWS_PER_W)
_GSEL = np.equal(((_rows // 2) % _G)[:, :, None],
                 np.arange(_G)[None, None, :]).astype(np.float32)


def kernel(router_outputs, attention_mask):
    del attention_mask
    p_parts, c_parts, z_parts = _router_loss_sc(router_outputs)
    p4 = p_parts.reshape(_NW, _ROWS_PER_W, _E, _E)
    c4 = c_parts.reshape(_NW, _ROWS_PER_W, _E, _E)
    probs = jnp.einsum("wjet,etE,wjg->gE", p4, _UNROT, _GSEL)
    counts = jnp.einsum("wjet,etE,wjg->gE", c4, _UNROT, _GSEL)
    z_loss = z_parts.sum() / (_G * _T)
    clipped = jnp.minimum(counts, _CAP)
    overflow = jnp.sum(counts - clipped, axis=1)
    final_c = clipped.at[:, 0].add(overflow)
    aux = jnp.mean((final_c / _T) * (probs / _T)) * _E ** 2
    return (0.001 * z_loss + 0.001 * aux).astype(jnp.float32)


# speedup vs baseline: 2.2787x; 1.9575x over previous
import functools

import jax
import jax.numpy as jnp
from jax import lax
from jax.experimental import pallas as pl
from jax.experimental.pallas import tpu as pltpu
from jax.experimental.pallas import tpu_sc as plsc

_E = 16
_G = 4
_T = 12 * 2048
_CAP = 2048.0
_ROW_TOKENS = 1024
_ROW_WORDS = _ROW_TOKENS * _E
_BLOCKS = _ROW_TOKENS // _E

_info = plsc.get_sparse_core_info()
_NC, _NS = _info.num_cores, _info.num_subcores
_NW = _NC * _NS
_RPW = 3
_ACC = _RPW * _E * _E

_LN2 = 0.6931471805599453
_SQRT2 = 1.4142135623730951


def _vlog(s):
    bits = lax.bitcast_convert_type(s, jnp.int32)
    e = jnp.right_shift(bits, 23) - 127
    mant = jnp.bitwise_or(jnp.bitwise_and(bits, 0x007FFFFF), 0x3F800000)
    f = lax.bitcast_convert_type(mant, jnp.float32)
    big = f >= _SQRT2
    f = jnp.where(big, f * 0.5, f)
    e = e + jnp.where(big, 1, 0)
    t = (f - 1.0) / (f + 1.0)
    t2 = t * t
    w = 2.0 * t * (1.0 + t2 * (1.0 / 3.0 + t2 * (0.2 + t2 * (1.0 / 7.0 + t2 / 9.0))))
    return e.astype(jnp.float32) * _LN2 + w


def _tree(f, xs):
    while len(xs) > 1:
        nxt = [f(xs[i], xs[i + 1]) for i in range(0, len(xs) - 1, 2)]
        if len(xs) % 2:
            nxt.append(xs[-1])
        xs = nxt
    return xs[0]


_EOFF = [(e // 8) * 8192 + (e % 8) * 128 for e in range(_E)]


def _block(buf, dynbase, zvec, off, pacc, cacc):
    l = [buf[pl.ds(_EOFF[e] + dynbase, _E)] for e in range(_E)]
    m = _tree(jnp.maximum, l)
    ex = [jnp.exp(le - m) for le in l]
    s = _tree(lambda a, b: a + b, ex)
    r = 1.0 / s
    for e in range(_E):
        plsc.addupdate(pacc.at[pl.ds(off + e * _E, _E)], ex[e] * r)
        plsc.addupdate(cacc.at[pl.ds(off + e * _E, _E)],
                       jnp.where(l[e] == m, 1.0, 0.0))
    logz = m + _vlog(s)
    return zvec + logz * logz


def _row_body(buf, j, pacc, cacc, zvec):
    off = j * _E * _E

    def body(k, z):
        dynbase = (k >> 3) * 1024 + (k & 7) * _E
        return _block(buf, dynbase, z, off, pacc, cacc)

    return lax.fori_loop(0, _BLOCKS, body, zvec)


@functools.partial(
    pl.kernel,
    out_type=(
        jax.ShapeDtypeStruct((_NW * _ACC,), jnp.float32),
        jax.ShapeDtypeStruct((_NW * _ACC,), jnp.float32),
        jax.ShapeDtypeStruct((_NW * _E,), jnp.float32),
    ),
    mesh=plsc.VectorSubcoreMesh(core_axis_name="c", subcore_axis_name="s"),
    compiler_params=pltpu.CompilerParams(needs_layout_passes=False,
                                         disable_bounds_checks=True,
                                         disable_semaphore_checks=True,
                                         skip_device_barrier=True),
    scratch_types=[
        pltpu.VMEM((_ROW_WORDS,), jnp.float32),
        pltpu.VMEM((_ROW_WORDS,), jnp.float32),
        pltpu.VMEM((_ACC,), jnp.float32),
        pltpu.VMEM((_ACC,), jnp.float32),
        pltpu.VMEM((_E,), jnp.float32),
        pltpu.SemaphoreType.DMA,
        pltpu.SemaphoreType.DMA,
    ],
)
def _router_loss_sc(x_hbm, p_out, c_out, z_out,
                    buf0, buf1, pacc, cacc, zacc, sem0, sem1):
    wid = lax.axis_index("s") * _NC + lax.axis_index("c")
    g = wid // 8
    q = wid % 8
    zero = jnp.zeros((_E,), jnp.float32)
    for i in range(_ACC // _E):
        pacc[pl.ds(i * _E, _E)] = zero
        cacc[pl.ds(i * _E, _E)] = zero

    def half_slab(n):
        m = q * _RPW + n
        i, h = m // 2, m % 2
        slab = (i * _G + g) * (2 * _ROW_WORDS)
        lo = slab + h * 8192
        return lo

    def copy_row(n, buf, sem):
        lo = half_slab(n)
        ch = pltpu.async_copy(x_hbm.at[pl.ds(lo, 8192)],
                              buf.at[pl.ds(0, 8192)], sem)
        cl = pltpu.async_copy(x_hbm.at[pl.ds(lo + 16384, 8192)],
                              buf.at[pl.ds(8192, 8192)], sem)
        return ch, cl

    cp0 = copy_row(0, buf0, sem0)
    cp1 = copy_row(1, buf1, sem1)
    cp0[0].wait()
    cp0[1].wait()
    z0 = _row_body(buf0, 0, pacc, cacc, zero)
    cp2 = copy_row(2, buf0, sem0)
    cp1[0].wait()
    cp1[1].wait()
    z1 = _row_body(buf1, 1, pacc, cacc, z0)
    cp2[0].wait()
    cp2[1].wait()
    z2 = _row_body(buf0, 2, pacc, cacc, z1)

    zacc[...] = z2
    pltpu.sync_copy(pacc, p_out.at[pl.ds(wid * _ACC, _ACC)])
    pltpu.sync_copy(cacc, c_out.at[pl.ds(wid * _ACC, _ACC)])
    pltpu.sync_copy(zacc, z_out.at[pl.ds(wid * _E, _E)])


def kernel(router_outputs, attention_mask):
    del attention_mask
    x = (router_outputs.transpose(0, 1, 3, 2)
         .reshape(12, _G, 2, 8, 16, 128)
         .transpose(0, 1, 2, 4, 3, 5)
         .reshape(-1))
    p_parts, c_parts, z_parts = _router_loss_sc(x)
    probs = p_parts.reshape(_G, 8 * _RPW, _E, _E).sum(axis=(1, 3))
    counts = c_parts.reshape(_G, 8 * _RPW, _E, _E).sum(axis=(1, 3))
    z_loss = z_parts.sum() / (_G * _T)
    clipped = jnp.minimum(counts, _CAP)
    overflow = jnp.sum(counts - clipped, axis=1)
    final_c = clipped.at[:, 0].add(overflow)
    aux = jnp.mean((final_c / _T) * (probs / _T)) * _E ** 2
    return (0.001 * z_loss + 0.001 * aux).astype(jnp.float32)
